# baseline (speedup 1.0000x reference)
import functools

import jax
import jax.numpy as jnp
import numpy as np
from jax import lax
from jax.experimental import pallas as pl
from jax.experimental.pallas import tpu as pltpu
from jax.experimental.pallas import tpu_sc as plsc

HIDDEN = 768
LANES = 16
NPAIR = HIDDEN // (2 * LANES)
EPS = 1e-5
N_TOKENS = 4 * 2048
NUM_WORKERS = 32
TOK_PER_WORKER = N_TOKENS // NUM_WORKERS
CHUNK = 32
NCHUNKS = TOK_PER_WORKER // CHUNK
ROW_BLOCK = 4096

HALF = HIDDEN // 2


def _gather_sum_body(ids_hbm, pids_hbm, word_hbm, pos_hbm, x_hbm,
                     idw_all, idp_all, w0, w1, p0, p1, xb0, xb1,
                     sw0, sw1, sp0, sp1, so0, so1):
    wid = lax.axis_index("s") * 2 + lax.axis_index("c")
    base = wid * TOK_PER_WORKER

    pltpu.sync_copy(ids_hbm.at[pl.ds(base, TOK_PER_WORKER)], idw_all)
    pltpu.sync_copy(pids_hbm.at[pl.ds(base, TOK_PER_WORKER)], idp_all)

    wr = (w0, w1)
    pr = (p0, p1)
    xb = (xb0, xb1)
    sw = (sw0, sw1)
    sp = (sp0, sp1)
    so = (so0, so1)

    gather_h = [None, None]
    out_h = [None, None]

    def start_gather(c):
        b = c % 2
        isl = pl.ds(c * CHUNK, CHUNK)
        hw = pltpu.async_copy(word_hbm.at[idw_all.at[isl]], wr[b], sw[b])
        hp = pltpu.async_copy(pos_hbm.at[idp_all.at[isl]], pr[b], sp[b])
        gather_h[b] = (hw, hp)

    start_gather(0)
    for c in range(NCHUNKS):
        b = c % 2
        hw, hp = gather_h[b]
        hw.wait()
        hp.wait()
        if c + 1 < NCHUNKS:
            if out_h[1 - b] is not None:
                out_h[1 - b].wait()
            start_gather(c + 1)

        wb, pb, ob = wr[b], pr[b], xb[b]

        @plsc.parallel_loop(0, CHUNK, unroll=4)
        def _sum_loop(t, wb=wb, pb=pb, ob=ob):
            for j in range(NPAIR):
                slu = pl.ds(j * LANES, LANES)
                slv = pl.ds(HALF + j * LANES, LANES)
                u = wb[t, slu] + pb[t, slu]
                v = wb[t, slv] + pb[t, slv]
                ui = plsc.bitcast(u, jnp.int32) + jnp.int32(0x8000)
                vi = plsc.bitcast(v, jnp.int32) + jnp.int32(0x8000)
                ob[t, pl.ds(j * LANES, LANES)] = (
                    lax.shift_right_logical(ui, 16)
                    | (vi & jnp.int32(-65536)))
        off = base + c * CHUNK
        out_h[b] = pltpu.async_copy(ob, x_hbm.at[pl.ds(off, CHUNK)], so[b])
    for b in (0, 1):
        if out_h[b] is not None:
            out_h[b].wait()


def _gather_sum(ids, pids, word_embeddings, position_embeddings):
    mesh = plsc.VectorSubcoreMesh(core_axis_name="c", subcore_axis_name="s")
    fn = functools.partial(
        pl.kernel,
        mesh=mesh,
        compiler_params=pltpu.CompilerParams(needs_layout_passes=False),
        out_type=jax.ShapeDtypeStruct((N_TOKENS, HALF), jnp.int32),
        scratch_types=[
            pltpu.VMEM((TOK_PER_WORKER,), jnp.int32),
            pltpu.VMEM((TOK_PER_WORKER,), jnp.int32),
            pltpu.VMEM((CHUNK, HIDDEN), jnp.float32),
            pltpu.VMEM((CHUNK, HIDDEN), jnp.float32),
            pltpu.VMEM((CHUNK, HIDDEN), jnp.float32),
            pltpu.VMEM((CHUNK, HIDDEN), jnp.float32),
            pltpu.VMEM((CHUNK, HALF), jnp.int32),
            pltpu.VMEM((CHUNK, HALF), jnp.int32),
            pltpu.SemaphoreType.DMA,
            pltpu.SemaphoreType.DMA,
            pltpu.SemaphoreType.DMA,
            pltpu.SemaphoreType.DMA,
            pltpu.SemaphoreType.DMA,
            pltpu.SemaphoreType.DMA,
        ],
    )(_gather_sum_body)
    return fn(ids, pids, word_embeddings, position_embeddings)


def _ln_body(tidf_ref, tt_ref, scale_ref, bias_ref, x_ref, o_ref):
    xi = x_ref[...]
    lo = lax.bitcast_convert_type(xi << 16, jnp.float32)
    hi = lax.bitcast_convert_type(xi & jnp.int32(-65536), jnp.float32)
    x = jnp.concatenate([lo, hi], axis=1)
    t0 = tt_ref[0:1, :]
    d = tt_ref[1:2, :] - t0
    x = x + t0 + tidf_ref[...] * d
    mean = jnp.mean(x, axis=1, keepdims=True)
    xc = x - mean
    var = jnp.mean(xc * xc, axis=1, keepdims=True)
    o_ref[...] = xc * lax.rsqrt(var + EPS) * scale_ref[...] + bias_ref[...]


def _type_ln(x, tidf, tt_p, scale_p, bias_p):
    grid = (N_TOKENS // ROW_BLOCK,)
    return pl.pallas_call(
        _ln_body,
        grid=grid,
        in_specs=[
            pl.BlockSpec((ROW_BLOCK, 1), lambda i: (i, 0)),
            pl.BlockSpec((2, HIDDEN), lambda i: (0, 0)),
            pl.BlockSpec((1, HIDDEN), lambda i: (0, 0)),
            pl.BlockSpec((1, HIDDEN), lambda i: (0, 0)),
            pl.BlockSpec((ROW_BLOCK, HALF), lambda i: (i, 0)),
        ],
        out_specs=pl.BlockSpec((ROW_BLOCK, HIDDEN), lambda i: (i, 0)),
        out_shape=jax.ShapeDtypeStruct((N_TOKENS, HIDDEN), jnp.float32),
    )(tidf, tt_p, scale_p, bias_p, x)


def kernel(input_ids, token_type_ids, position_ids, attention_mask,
           word_embeddings, position_embeddings, token_type_embeddings,
           ln_scale, ln_bias):
    del attention_mask
    ids = input_ids.reshape(-1).astype(jnp.int32)
    pids = position_ids.reshape(-1).astype(jnp.int32)
    tidf = token_type_ids.reshape(-1, 1).astype(jnp.float32)
    x = _gather_sum(ids, pids, word_embeddings, position_embeddings)
    out = _type_ln(x, tidf, token_type_embeddings,
                   ln_scale.reshape(1, HIDDEN), ln_bias.reshape(1, HIDDEN))
    return out.reshape(input_ids.shape + (HIDDEN,))


# speedup vs baseline: 1.0154x; 1.0154x over previous
import functools

import jax
import jax.numpy as jnp
import numpy as np
from jax import lax
from jax.experimental import pallas as pl
from jax.experimental.pallas import tpu as pltpu
from jax.experimental.pallas import tpu_sc as plsc

HIDDEN = 768
LANES = 16
NPAIR = HIDDEN // (2 * LANES)
EPS = 1e-5
N_TOKENS = 4 * 2048
NUM_WORKERS = 32
TOK_PER_WORKER = N_TOKENS // NUM_WORKERS
CHUNK = 32
NCHUNKS = TOK_PER_WORKER // CHUNK
ROW_BLOCK = 2048

HALF = HIDDEN // 2


def _gather_sum_body(ids_hbm, pids_hbm, word_hbm, pos_hbm, x_hbm,
                     idw_all, idp_all, w0, w1, p0, p1, xb0, xb1,
                     sw0, sw1, sp0, sp1, so0, so1):
    wid = lax.axis_index("s") * 2 + lax.axis_index("c")
    base = wid * TOK_PER_WORKER

    pltpu.sync_copy(ids_hbm.at[pl.ds(base, TOK_PER_WORKER)], idw_all)
    pltpu.sync_copy(pids_hbm.at[pl.ds(base, TOK_PER_WORKER)], idp_all)

    wr = (w0, w1)
    pr = (p0, p1)
    xb = (xb0, xb1)
    sw = (sw0, sw1)
    sp = (sp0, sp1)
    so = (so0, so1)

    gather_h = [None, None]
    out_h = [None, None]

    def start_gather(c):
        b = c % 2
        isl = pl.ds(c * CHUNK, CHUNK)
        hw = pltpu.async_copy(word_hbm.at[idw_all.at[isl]], wr[b], sw[b])
        hp = pltpu.async_copy(pos_hbm.at[idp_all.at[isl]], pr[b], sp[b])
        gather_h[b] = (hw, hp)

    start_gather(0)
    for c in range(NCHUNKS):
        b = c % 2
        hw, hp = gather_h[b]
        hw.wait()
        hp.wait()
        if c + 1 < NCHUNKS:
            if out_h[1 - b] is not None:
                out_h[1 - b].wait()
            start_gather(c + 1)

        wb, pb, ob = wr[b], pr[b], xb[b]

        @plsc.parallel_loop(0, CHUNK, unroll=4)
        def _sum_loop(t, wb=wb, pb=pb, ob=ob):
            for j in range(NPAIR):
                slu = pl.ds(j * LANES, LANES)
                slv = pl.ds(HALF + j * LANES, LANES)
                u = wb[t, slu] + pb[t, slu]
                v = wb[t, slv] + pb[t, slv]
                ui = plsc.bitcast(u, jnp.int32) + jnp.int32(0x8000)
                vi = plsc.bitcast(v, jnp.int32) + jnp.int32(0x8000)
                ob[t, pl.ds(j * LANES, LANES)] = (
                    lax.shift_right_logical(ui, 16)
                    | (vi & jnp.int32(-65536)))
        off = base + c * CHUNK
        out_h[b] = pltpu.async_copy(ob, x_hbm.at[pl.ds(off, CHUNK)], so[b])
    for b in (0, 1):
        if out_h[b] is not None:
            out_h[b].wait()


def _gather_sum(ids, pids, word_embeddings, position_embeddings):
    mesh = plsc.VectorSubcoreMesh(core_axis_name="c", subcore_axis_name="s")
    fn = functools.partial(
        pl.kernel,
        mesh=mesh,
        compiler_params=pltpu.CompilerParams(needs_layout_passes=False),
        out_type=jax.ShapeDtypeStruct((N_TOKENS, HALF), jnp.int32),
        scratch_types=[
            pltpu.VMEM((TOK_PER_WORKER,), jnp.int32),
            pltpu.VMEM((TOK_PER_WORKER,), jnp.int32),
            pltpu.VMEM((CHUNK, HIDDEN), jnp.float32),
            pltpu.VMEM((CHUNK, HIDDEN), jnp.float32),
            pltpu.VMEM((CHUNK, HIDDEN), jnp.float32),
            pltpu.VMEM((CHUNK, HIDDEN), jnp.float32),
            pltpu.VMEM((CHUNK, HALF), jnp.int32),
            pltpu.VMEM((CHUNK, HALF), jnp.int32),
            pltpu.SemaphoreType.DMA,
            pltpu.SemaphoreType.DMA,
            pltpu.SemaphoreType.DMA,
            pltpu.SemaphoreType.DMA,
            pltpu.SemaphoreType.DMA,
            pltpu.SemaphoreType.DMA,
        ],
    )(_gather_sum_body)
    return fn(ids, pids, word_embeddings, position_embeddings)


def _ln_body(tidf_ref, tt_ref, scale_ref, bias_ref, x_ref, o_ref):
    xi = x_ref[...]
    lo = lax.bitcast_convert_type(xi << 16, jnp.float32)
    hi = lax.bitcast_convert_type(xi & jnp.int32(-65536), jnp.float32)
    x = jnp.concatenate([lo, hi], axis=1)
    t0 = tt_ref[0:1, :]
    d = tt_ref[1:2, :] - t0
    x = x + t0 + tidf_ref[...] * d
    mean = jnp.mean(x, axis=1, keepdims=True)
    xc = x - mean
    var = jnp.mean(xc * xc, axis=1, keepdims=True)
    o_ref[...] = xc * lax.rsqrt(var + EPS) * scale_ref[...] + bias_ref[...]


def _type_ln(x, tidf, tt_p, scale_p, bias_p):
    grid = (N_TOKENS // ROW_BLOCK,)
    return pl.pallas_call(
        _ln_body,
        grid=grid,
        in_specs=[
            pl.BlockSpec((ROW_BLOCK, 1), lambda i: (i, 0)),
            pl.BlockSpec((2, HIDDEN), lambda i: (0, 0)),
            pl.BlockSpec((1, HIDDEN), lambda i: (0, 0)),
            pl.BlockSpec((1, HIDDEN), lambda i: (0, 0)),
            pl.BlockSpec((ROW_BLOCK, HALF), lambda i: (i, 0)),
        ],
        out_specs=pl.BlockSpec((ROW_BLOCK, HIDDEN), lambda i: (i, 0)),
        out_shape=jax.ShapeDtypeStruct((N_TOKENS, HIDDEN), jnp.float32),
    )(tidf, tt_p, scale_p, bias_p, x)


def kernel(input_ids, token_type_ids, position_ids, attention_mask,
           word_embeddings, position_embeddings, token_type_embeddings,
           ln_scale, ln_bias):
    del attention_mask
    ids = input_ids.reshape(-1).astype(jnp.int32)
    pids = position_ids.reshape(-1).astype(jnp.int32)
    tidf = token_type_ids.reshape(-1, 1).astype(jnp.float32)
    x = _gather_sum(ids, pids, word_embeddings, position_embeddings)
    out = _type_ln(x, tidf, token_type_embeddings,
                   ln_scale.reshape(1, HIDDEN), ln_bias.reshape(1, HIDDEN))
    return out.reshape(input_ids.shape + (HIDDEN,))
